# baseline (speedup 1.0000x reference)
import jax
import jax.numpy as jnp
from jax.experimental import pallas as pl
from jax.experimental.pallas import tpu as pltpu


def _linear_body(v_ref, wt_ref, b_ref, o_ref):
    v = v_ref[...]
    wt = wt_ref[...]
    o_ref[...] = (
        jax.lax.dot_general(v, wt, (((1,), (0,)), ((), ())),
                            preferred_element_type=jnp.float32)
        + b_ref[...]
    )


def kernel(values, offsets, W, b):
    del offsets
    T, IN_F = values.shape
    OUT_F = W.shape[0]
    wt = W.T
    b2 = b.reshape(1, OUT_F)
    BT = 2048
    grid = (T // BT,)
    out = pl.pallas_call(
        _linear_body,
        grid=grid,
        in_specs=[
            pl.BlockSpec((BT, IN_F), lambda i: (i, 0)),
            pl.BlockSpec((IN_F, OUT_F), lambda i: (0, 0)),
            pl.BlockSpec((1, OUT_F), lambda i: (0, 0)),
        ],
        out_specs=pl.BlockSpec((BT, OUT_F), lambda i: (i, 0)),
        out_shape=jax.ShapeDtypeStruct((T, OUT_F), jnp.float32),
        compiler_params=pltpu.CompilerParams(
            dimension_semantics=("arbitrary",),
        ),
    )(values, wt, b2)
    return out
